# baseline (speedup 1.0000x reference)
import functools

import jax
import jax.numpy as jnp
from jax import lax
from jax.experimental import pallas as pl
from jax.experimental.pallas import tpu as pltpu
from jax.experimental.pallas import tpu_sc as plsc

N = 50000
E = 800000
F_IN = 4
HID = 32
T = 8

NC = 2
NS = 16
NW = NC * NS
K = 128
SK = 12
SCH = 18
CH = SK * SCH
EPW = K * CH
EPAD = NW * EPW
NP = 50048
RPS = NP // NS

_MESH = plsc.VectorSubcoreMesh(core_axis_name="c", subcore_axis_name="s")
_SC_PARAMS = pltpu.CompilerParams(use_tc_tiling_on_sc=False)


def _spmm_body(xs_hbm, srcr_hbm, dstr_hbm, ewr_hbm, zero_hbm, out0_hbm, out1_hbm,
               sb0, db0, wb0, sb1, db1, wb1, rows0, rows1, acc_sh, esem, gsem):
    c = lax.axis_index("c")
    s = lax.axis_index("s")
    wid = c * NS + s
    base_row = wid * SCH
    pltpu.sync_copy(zero_hbm, acc_sh.at[pl.ds(s * RPS, RPS)])
    pltpu.sync_copy(srcr_hbm.at[base_row], sb0)
    pltpu.sync_copy(dstr_hbm.at[base_row], db0)
    pltpu.sync_copy(ewr_hbm.at[base_row], wb0)
    plsc.subcore_barrier()

    def scale_scatter(rows, wb, db, k):
        for j16 in range(K // 16):
            e16 = wb[k, pl.ds(j16 * 16, 16)]
            for j in range(16):
                e = e16[j]
                row = j16 * 16 + j
                rows[row, pl.ds(0, 16)] = rows[row, pl.ds(0, 16)] * e
                rows[row, pl.ds(16, 16)] = rows[row, pl.ds(16, 16)] * e
        pltpu.sync_copy(rows, acc_sh.at[db.at[k]], add=True)

    def wait_rows(rows):
        pltpu.make_async_copy(xs_hbm.at[pl.ds(0, K)], rows, gsem).wait()

    def super_body(G, sb, db, wb, sbn, dbn, wbn):
        nxt = base_row + jnp.minimum(G + 1, SCH - 1)
        pltpu.async_copy(srcr_hbm.at[nxt], sbn, esem)
        pltpu.async_copy(dstr_hbm.at[nxt], dbn, esem)
        pltpu.async_copy(ewr_hbm.at[nxt], wbn, esem)
        pltpu.async_copy(xs_hbm.at[sb.at[0]], rows0, gsem)

        def body(k2, carry):
            k = 2 * k2
            wait_rows(rows0)
            pltpu.async_copy(xs_hbm.at[sb.at[k + 1]], rows1, gsem)
            scale_scatter(rows0, wb, db, k)
            wait_rows(rows1)

            @pl.when(k2 < SK // 2 - 1)
            def _():
                pltpu.async_copy(xs_hbm.at[sb.at[k + 2]], rows0, gsem)

            scale_scatter(rows1, wb, db, k + 1)
            return carry

        lax.fori_loop(0, SK // 2, body, 0)
        pltpu.make_async_copy(srcr_hbm.at[base_row], sbn, esem).wait()
        pltpu.make_async_copy(dstr_hbm.at[base_row], dbn, esem).wait()
        pltpu.make_async_copy(ewr_hbm.at[base_row], wbn, esem).wait()

    def pair(i, carry):
        super_body(2 * i, sb0, db0, wb0, sb1, db1, wb1)
        super_body(2 * i + 1, sb1, db1, wb1, sb0, db0, wb0)
        return carry

    lax.fori_loop(0, SCH // 2, pair, 0)
    plsc.subcore_barrier()

    @pl.when(c == 0)
    def _():
        pltpu.sync_copy(acc_sh.at[pl.ds(s * RPS, RPS)],
                        out0_hbm.at[pl.ds(s * RPS, RPS)])

    @pl.when(c == 1)
    def _():
        pltpu.sync_copy(acc_sh.at[pl.ds(s * RPS, RPS)],
                        out1_hbm.at[pl.ds(s * RPS, RPS)])


_spmm_call = functools.partial(
    pl.kernel,
    out_type=[jax.ShapeDtypeStruct((NP, HID), jnp.float32),
              jax.ShapeDtypeStruct((NP, HID), jnp.float32)],
    mesh=_MESH,
    scratch_types=[
        pltpu.VMEM((SK, K), jnp.int32),
        pltpu.VMEM((SK, K), jnp.int32),
        pltpu.VMEM((SK, K), jnp.float32),
        pltpu.VMEM((SK, K), jnp.int32),
        pltpu.VMEM((SK, K), jnp.int32),
        pltpu.VMEM((SK, K), jnp.float32),
        pltpu.VMEM((K, HID), jnp.float32),
        pltpu.VMEM((K, HID), jnp.float32),
        pltpu.VMEM_SHARED((NP, HID), jnp.float32),
        pltpu.SemaphoreType.DMA,
        pltpu.SemaphoreType.DMA,
    ],
    compiler_params=_SC_PARAMS,
)(_spmm_body)


_B = 5000
_G = N // _B

def _row_spec():
    return pl.BlockSpec((_B, HID), lambda i: (i, 0))

def _full_spec(r, c):
    return pl.BlockSpec((r, c), lambda i: (0, 0))


def _prep_body(deg0, deg1, xflat, dis, xs):
    di = lax.rsqrt(deg0[:, :] + deg1[:, :])
    dis[:, :] = di
    xs[:, :] = xflat[:, :] * di


def _prep(deg0, deg1, xflat):
    return pl.pallas_call(
        _prep_body,
        grid=(_G,),
        in_specs=[_row_spec(), _row_spec(), _row_spec()],
        out_specs=[_row_spec(), _row_spec()],
        out_shape=[jax.ShapeDtypeStruct((N, HID), jnp.float32)] * 2,
    )(deg0, deg1, xflat)


def _mm(a, w):
    return jnp.dot(a, w[:, :], preferred_element_type=jnp.float32)


def _axt_of(ax0, ax1, dis, t):
    lo = t * F_IN
    return dis[:, lo:lo + F_IN] * (ax0[:, lo:lo + F_IN] + ax1[:, lo:lo + F_IN])


def _t0_body(ax0, ax1, dis, Wzx, bz, Whx, bh, h, hs):
    di = dis[:, :]
    AX0 = di[:, 0:F_IN] * (ax0[:, 0:F_IN] + ax1[:, 0:F_IN])
    z = jax.nn.sigmoid(_mm(AX0, Wzx) + bz[:, :])
    hc = jnp.tanh(_mm(AX0, Whx) + bh[:, :])
    hn = (1.0 - z) * hc
    h[:, :] = hn
    hs[:, :] = hn * di


def _t0(ax0, ax1, dis, Wzx, bz, Whx, bh):
    return pl.pallas_call(
        _t0_body,
        grid=(_G,),
        in_specs=[_row_spec(), _row_spec(), _row_spec(),
                  _full_spec(F_IN, HID), _full_spec(1, HID),
                  _full_spec(F_IN, HID), _full_spec(1, HID)],
        out_specs=[_row_spec(), _row_spec()],
        out_shape=[jax.ShapeDtypeStruct((N, HID), jnp.float32)] * 2,
    )(ax0, ax1, dis, Wzx, bz, Whx, bh)


def _make_gates(t):
    def body(p0, p1, h, ax0, ax1, dis, Wzx, Wzh, bz, Wrx, Wrh, br,
             z_o, rh_o, rhs_o):
        di = dis[:, :]
        P = di * (p0[:, :] + p1[:, :])
        lo = t * F_IN
        AXt = di[:, lo:lo + F_IN] * (ax0[:, lo:lo + F_IN] + ax1[:, lo:lo + F_IN])
        z = jax.nn.sigmoid(_mm(AXt, Wzx) + _mm(P, Wzh) + bz[:, :])
        r = jax.nn.sigmoid(_mm(AXt, Wrx) + _mm(P, Wrh) + br[:, :])
        rh = r * h[:, :]
        z_o[:, :] = z
        rh_o[:, :] = rh
        rhs_o[:, :] = rh * di

    def call(p0, p1, h, ax0, ax1, dis, Wzx, Wzh, bz, Wrx, Wrh, br):
        return pl.pallas_call(
            body,
            grid=(_G,),
            in_specs=[_row_spec()] * 6 +
                     [_full_spec(F_IN, HID), _full_spec(HID, HID), _full_spec(1, HID),
                      _full_spec(F_IN, HID), _full_spec(HID, HID), _full_spec(1, HID)],
            out_specs=[_row_spec()] * 3,
            out_shape=[jax.ShapeDtypeStruct((N, HID), jnp.float32)] * 3,
        )(p0, p1, h, ax0, ax1, dis, Wzx, Wzh, bz, Wrx, Wrh, br)

    return call


def _make_update(t):
    def body(q0, q1, rh, z, h, ax0, ax1, dis, Whx, Whh, bh, h_o, hs_o):
        di = dis[:, :]
        Q = di * (q0[:, :] + q1[:, :])
        lo = t * F_IN
        AXt = di[:, lo:lo + F_IN] * (ax0[:, lo:lo + F_IN] + ax1[:, lo:lo + F_IN])
        hc = jnp.tanh(_mm(AXt, Whx) + _mm(Q, Whh) + bh[:, :])
        zz = z[:, :]
        hn = zz * h[:, :] + (1.0 - zz) * hc
        h_o[:, :] = hn
        hs_o[:, :] = hn * di

    def call(q0, q1, rh, z, h, ax0, ax1, dis, Whx, Whh, bh):
        return pl.pallas_call(
            body,
            grid=(_G,),
            in_specs=[_row_spec()] * 8 +
                     [_full_spec(F_IN, HID), _full_spec(HID, HID), _full_spec(1, HID)],
            out_specs=[_row_spec()] * 2,
            out_shape=[jax.ShapeDtypeStruct((N, HID), jnp.float32)] * 2,
        )(q0, q1, rh, z, h, ax0, ax1, dis, Whx, Whh, bh)

    return call


def _attn_body(*refs):
    hs = refs[:T]
    Wa, ba, ctxT, WfT, bf = refs[T:T + 5]
    out = refs[T + 5]
    als = []
    for t in range(T):
        S = jnp.tanh(_mm(hs[t][:, :], Wa) + ba[:, :])
        als.append(jnp.sum(S * ctxT[:, :], axis=1, keepdims=True))
    al = jnp.concatenate(als, axis=1)
    m = jnp.max(al, axis=1, keepdims=True)
    ex = jnp.exp(al - m)
    ssum = jnp.sum(ex, axis=1, keepdims=True)
    ctxv = jnp.zeros_like(hs[0][:, :])
    for t in range(T):
        ctxv = ctxv + (ex[:, t:t + 1] / ssum) * hs[t][:, :]
    o = jnp.sum(ctxv * WfT[:, :], axis=1, keepdims=True) + bf[:, :]
    out[:, :] = jnp.broadcast_to(o, (o.shape[0], HID))


def _attn(hs, Wa, ba_row, ctxT, WfT, bf_row):
    res = pl.pallas_call(
        _attn_body,
        grid=(_G,),
        in_specs=[_row_spec()] * T + [_full_spec(HID, HID), _full_spec(1, HID),
                                      _full_spec(1, HID), _full_spec(1, HID),
                                      _full_spec(1, 1)],
        out_specs=[_row_spec()],
        out_shape=[jax.ShapeDtypeStruct((N, HID), jnp.float32)],
        compiler_params=pltpu.CompilerParams(vmem_limit_bytes=100 * 1024 * 1024),
    )(*hs, Wa, ba_row, ctxT, WfT, bf_row)[0]
    return res[:, 0:1]


def kernel(x, edge_index, edge_weight, Wz, bz, Wr, br, Wh, bh, Wa, ba, ctx, Wf, bf):
    pad = EPAD - E - N
    loop = jnp.arange(N, dtype=jnp.int32)
    srcr = jnp.concatenate([edge_index[0], loop, jnp.zeros((pad,), jnp.int32)]
                           ).reshape(NW * SCH, SK, K)
    dstr = jnp.concatenate([edge_index[1], loop, jnp.zeros((pad,), jnp.int32)]
                           ).reshape(NW * SCH, SK, K)
    ewr = jnp.concatenate([edge_weight, jnp.ones((N,), jnp.float32),
                           jnp.zeros((pad,), jnp.float32)]).reshape(NW * SCH, SK, K)
    zeros32 = jnp.zeros((RPS, HID), jnp.float32)
    ones32 = jnp.ones((N, HID), jnp.float32)
    xflat = jnp.transpose(x, (0, 2, 1)).reshape(N, T * F_IN)

    Wzx, Wzh = Wz[:F_IN], Wz[F_IN:]
    Wrx, Wrh = Wr[:F_IN], Wr[F_IN:]
    Whx, Whh = Wh[:F_IN], Wh[F_IN:]
    bz_r = bz.reshape(1, HID)
    br_r = br.reshape(1, HID)
    bh_r = bh.reshape(1, HID)
    ba_r = ba.reshape(1, HID)
    ctxT = ctx.reshape(1, HID)
    WfT = Wf.reshape(1, HID)
    bf_r = bf.reshape(1, 1)

    deg0, deg1 = _spmm_call(ones32, srcr, dstr, ewr, zeros32)
    dis, xs = _prep(deg0, deg1, xflat)

    ax0, ax1 = _spmm_call(xs, srcr, dstr, ewr, zeros32)

    h, hs = _t0(ax0, ax1, dis, Wzx, bz_r, Whx, bh_r)
    hidden = [h]
    for t in range(1, T):
        p0, p1 = _spmm_call(hs, srcr, dstr, ewr, zeros32)
        z, rh, rhs = _make_gates(t)(p0, p1, h, ax0, ax1, dis,
                                    Wzx, Wzh, bz_r, Wrx, Wrh, br_r)
        q0, q1 = _spmm_call(rhs, srcr, dstr, ewr, zeros32)
        h, hs = _make_update(t)(q0, q1, rh, z, h, ax0, ax1, dis,
                                Whx, Whh, bh_r)
        hidden.append(h)

    return _attn(hidden, Wa, ba_r, ctxT, WfT, bf_r)


# speedup vs baseline: 2.0488x; 2.0488x over previous
import functools

import jax
import jax.numpy as jnp
from jax import lax
from jax.experimental import pallas as pl
from jax.experimental.pallas import tpu as pltpu
from jax.experimental.pallas import tpu_sc as plsc

N = 50000
E = 800000
F_IN = 4
HID = 32
T = 8

NC = 2
NS = 16
NW = NC * NS
K = 128
SK = 12
SCH = 18
CH = SK * SCH
EPW = K * CH
EPAD = NW * EPW
NP = 50048
RPS = NP // NS

_MESH = plsc.VectorSubcoreMesh(core_axis_name="c", subcore_axis_name="s")
_SC_PARAMS = pltpu.CompilerParams(use_tc_tiling_on_sc=False)


def _spmm_body(xs_hbm, srcr_hbm, dstr_hbm, ewr_hbm, zero_hbm, out0_hbm, out1_hbm,
               sb0, db0, wb0, sb1, db1, wb1, rows0, rows1, acc_sh, esem, gsem):
    c = lax.axis_index("c")
    s = lax.axis_index("s")
    wid = c * NS + s
    base_row = wid * SCH
    pltpu.sync_copy(zero_hbm, acc_sh.at[pl.ds(s * RPS, RPS)])
    pltpu.sync_copy(srcr_hbm.at[base_row], sb0)
    pltpu.sync_copy(dstr_hbm.at[base_row], db0)
    pltpu.sync_copy(ewr_hbm.at[base_row], wb0)
    plsc.subcore_barrier()

    def scale_scatter(rows, wb, db, k):
        for j16 in range(K // 16):
            e16 = wb[k, pl.ds(j16 * 16, 16)]
            for j in range(16):
                e = e16[j]
                row = j16 * 16 + j
                rows[row, pl.ds(0, 16)] = rows[row, pl.ds(0, 16)] * e
                rows[row, pl.ds(16, 16)] = rows[row, pl.ds(16, 16)] * e
        pltpu.sync_copy(rows, acc_sh.at[db.at[k]], add=True)

    def wait_rows(rows):
        pltpu.make_async_copy(xs_hbm.at[pl.ds(0, K)], rows, gsem).wait()

    def super_body(G, sb, db, wb, sbn, dbn, wbn):
        nxt = base_row + jnp.minimum(G + 1, SCH - 1)
        pltpu.async_copy(srcr_hbm.at[nxt], sbn, esem)
        pltpu.async_copy(dstr_hbm.at[nxt], dbn, esem)
        pltpu.async_copy(ewr_hbm.at[nxt], wbn, esem)
        pltpu.async_copy(xs_hbm.at[sb.at[0]], rows0, gsem)

        def body(k2, carry):
            k = 2 * k2
            wait_rows(rows0)
            pltpu.async_copy(xs_hbm.at[sb.at[k + 1]], rows1, gsem)
            scale_scatter(rows0, wb, db, k)
            wait_rows(rows1)

            @pl.when(k2 < SK // 2 - 1)
            def _():
                pltpu.async_copy(xs_hbm.at[sb.at[k + 2]], rows0, gsem)

            scale_scatter(rows1, wb, db, k + 1)
            return carry

        lax.fori_loop(0, SK // 2, body, 0)
        pltpu.make_async_copy(srcr_hbm.at[base_row], sbn, esem).wait()
        pltpu.make_async_copy(dstr_hbm.at[base_row], dbn, esem).wait()
        pltpu.make_async_copy(ewr_hbm.at[base_row], wbn, esem).wait()

    def pair(i, carry):
        super_body(2 * i, sb0, db0, wb0, sb1, db1, wb1)
        super_body(2 * i + 1, sb1, db1, wb1, sb0, db0, wb0)
        return carry

    lax.fori_loop(0, SCH // 2, pair, 0)
    plsc.subcore_barrier()

    @pl.when(c == 0)
    def _():
        pltpu.sync_copy(acc_sh.at[pl.ds(s * RPS, RPS)],
                        out0_hbm.at[pl.ds(s * RPS, RPS)])

    @pl.when(c == 1)
    def _():
        pltpu.sync_copy(acc_sh.at[pl.ds(s * RPS, RPS)],
                        out1_hbm.at[pl.ds(s * RPS, RPS)])


_spmm_call = functools.partial(
    pl.kernel,
    out_type=[jax.ShapeDtypeStruct((NP, HID), jnp.float32),
              jax.ShapeDtypeStruct((NP, HID), jnp.float32)],
    mesh=_MESH,
    scratch_types=[
        pltpu.VMEM((SK, K), jnp.int32),
        pltpu.VMEM((SK, K), jnp.int32),
        pltpu.VMEM((SK, K), jnp.float32),
        pltpu.VMEM((SK, K), jnp.int32),
        pltpu.VMEM((SK, K), jnp.int32),
        pltpu.VMEM((SK, K), jnp.float32),
        pltpu.VMEM((K, HID), jnp.float32),
        pltpu.VMEM((K, HID), jnp.float32),
        pltpu.VMEM_SHARED((NP, HID), jnp.float32),
        pltpu.SemaphoreType.DMA,
        pltpu.SemaphoreType.DMA,
    ],
    compiler_params=_SC_PARAMS,
)(_spmm_body)


_B = 5000
_G = N // _B

def _row_spec():
    return pl.BlockSpec((_B, HID), lambda i: (i, 0))

def _full_spec(r, c):
    return pl.BlockSpec((r, c), lambda i: (0, 0))


def _prep_body(deg0, deg1, xflat, dis, xs):
    di = lax.rsqrt(deg0[:, :] + deg1[:, :])
    dis[:, :] = di
    xs[:, :] = xflat[:, :] * di


def _prep(deg0, deg1, xflat):
    return pl.pallas_call(
        _prep_body,
        grid=(_G,),
        in_specs=[_row_spec(), _row_spec(), _row_spec()],
        out_specs=[_row_spec(), _row_spec()],
        out_shape=[jax.ShapeDtypeStruct((N, HID), jnp.float32)] * 2,
    )(deg0, deg1, xflat)


def _mm(a, w):
    return jnp.dot(a, w[:, :], preferred_element_type=jnp.float32)


def _axt_of(ax0, ax1, dis, t):
    lo = t * F_IN
    return dis[:, lo:lo + F_IN] * (ax0[:, lo:lo + F_IN] + ax1[:, lo:lo + F_IN])


def _t0_body(ax0, ax1, dis, Wzx, bz, Whx, bh, h, hs):
    di = dis[:, :]
    AX0 = di[:, 0:F_IN] * (ax0[:, 0:F_IN] + ax1[:, 0:F_IN])
    z = jax.nn.sigmoid(_mm(AX0, Wzx) + bz[:, :])
    hc = jnp.tanh(_mm(AX0, Whx) + bh[:, :])
    hn = (1.0 - z) * hc
    h[:, :] = hn
    hs[:, :] = hn * di


def _t0(ax0, ax1, dis, Wzx, bz, Whx, bh):
    return pl.pallas_call(
        _t0_body,
        grid=(_G,),
        in_specs=[_row_spec(), _row_spec(), _row_spec(),
                  _full_spec(F_IN, HID), _full_spec(1, HID),
                  _full_spec(F_IN, HID), _full_spec(1, HID)],
        out_specs=[_row_spec(), _row_spec()],
        out_shape=[jax.ShapeDtypeStruct((N, HID), jnp.float32)] * 2,
    )(ax0, ax1, dis, Wzx, bz, Whx, bh)


def _make_gates(t):
    def body(p0, p1, h, ax0, ax1, dis, Wzx, Wzh, bz, Wrx, Wrh, br,
             z_o, rh_o, rhs_o):
        di = dis[:, :]
        P = di * (p0[:, :] + p1[:, :])
        lo = t * F_IN
        AXt = di[:, lo:lo + F_IN] * (ax0[:, lo:lo + F_IN] + ax1[:, lo:lo + F_IN])
        z = jax.nn.sigmoid(_mm(AXt, Wzx) + _mm(P, Wzh) + bz[:, :])
        r = jax.nn.sigmoid(_mm(AXt, Wrx) + _mm(P, Wrh) + br[:, :])
        rh = r * h[:, :]
        z_o[:, :] = z
        rh_o[:, :] = rh
        rhs_o[:, :] = rh * di

    def call(p0, p1, h, ax0, ax1, dis, Wzx, Wzh, bz, Wrx, Wrh, br):
        return pl.pallas_call(
            body,
            grid=(_G,),
            in_specs=[_row_spec()] * 6 +
                     [_full_spec(F_IN, HID), _full_spec(HID, HID), _full_spec(1, HID),
                      _full_spec(F_IN, HID), _full_spec(HID, HID), _full_spec(1, HID)],
            out_specs=[_row_spec()] * 3,
            out_shape=[jax.ShapeDtypeStruct((N, HID), jnp.float32)] * 3,
        )(p0, p1, h, ax0, ax1, dis, Wzx, Wzh, bz, Wrx, Wrh, br)

    return call


def _make_update(t):
    def body(q0, q1, rh, z, h, ax0, ax1, dis, Whx, Whh, bh, h_o, hs_o):
        di = dis[:, :]
        Q = di * (q0[:, :] + q1[:, :])
        lo = t * F_IN
        AXt = di[:, lo:lo + F_IN] * (ax0[:, lo:lo + F_IN] + ax1[:, lo:lo + F_IN])
        hc = jnp.tanh(_mm(AXt, Whx) + _mm(Q, Whh) + bh[:, :])
        zz = z[:, :]
        hn = zz * h[:, :] + (1.0 - zz) * hc
        h_o[:, :] = hn
        hs_o[:, :] = hn * di

    def call(q0, q1, rh, z, h, ax0, ax1, dis, Whx, Whh, bh):
        return pl.pallas_call(
            body,
            grid=(_G,),
            in_specs=[_row_spec()] * 8 +
                     [_full_spec(F_IN, HID), _full_spec(HID, HID), _full_spec(1, HID)],
            out_specs=[_row_spec()] * 2,
            out_shape=[jax.ShapeDtypeStruct((N, HID), jnp.float32)] * 2,
        )(q0, q1, rh, z, h, ax0, ax1, dis, Whx, Whh, bh)

    return call


def _attn_body(*refs):
    hs = refs[:T]
    Wa, ba, ctxT, WfT, bf = refs[T:T + 5]
    out = refs[T + 5]
    als = []
    for t in range(T):
        S = jnp.tanh(_mm(hs[t][:, :], Wa) + ba[:, :])
        als.append(jnp.sum(S * ctxT[:, :], axis=1, keepdims=True))
    al = jnp.concatenate(als, axis=1)
    m = jnp.max(al, axis=1, keepdims=True)
    ex = jnp.exp(al - m)
    ssum = jnp.sum(ex, axis=1, keepdims=True)
    ctxv = jnp.zeros_like(hs[0][:, :])
    for t in range(T):
        ctxv = ctxv + (ex[:, t:t + 1] / ssum) * hs[t][:, :]
    o = jnp.sum(ctxv * WfT[:, :], axis=1, keepdims=True) + bf[:, :]
    out[:, :] = jnp.broadcast_to(o, (o.shape[0], HID))


def _attn(hs, Wa, ba_row, ctxT, WfT, bf_row):
    res = pl.pallas_call(
        _attn_body,
        grid=(_G,),
        in_specs=[_row_spec()] * T + [_full_spec(HID, HID), _full_spec(1, HID),
                                      _full_spec(1, HID), _full_spec(1, HID),
                                      _full_spec(1, 1)],
        out_specs=[_row_spec()],
        out_shape=[jax.ShapeDtypeStruct((N, HID), jnp.float32)],
        compiler_params=pltpu.CompilerParams(vmem_limit_bytes=100 * 1024 * 1024),
    )(*hs, Wa, ba_row, ctxT, WfT, bf_row)[0]
    return res[:, 0:1]


def kernel(x, edge_index, edge_weight, Wz, bz, Wr, br, Wh, bh, Wa, ba, ctx, Wf, bf):
    pad = EPAD - E - N
    loop = jnp.arange(N, dtype=jnp.int32)
    padidx = jnp.arange(pad, dtype=jnp.int32) % N
    srcr = jnp.concatenate([edge_index[0], loop, padidx]).reshape(NW * SCH, SK, K)
    dstr = jnp.concatenate([edge_index[1], loop, padidx]).reshape(NW * SCH, SK, K)
    ewr = jnp.concatenate([edge_weight, jnp.ones((N,), jnp.float32),
                           jnp.zeros((pad,), jnp.float32)]).reshape(NW * SCH, SK, K)
    zeros32 = jnp.zeros((RPS, HID), jnp.float32)
    ones32 = jnp.ones((N, HID), jnp.float32)
    xflat = jnp.transpose(x, (0, 2, 1)).reshape(N, T * F_IN)

    Wzx, Wzh = Wz[:F_IN], Wz[F_IN:]
    Wrx, Wrh = Wr[:F_IN], Wr[F_IN:]
    Whx, Whh = Wh[:F_IN], Wh[F_IN:]
    bz_r = bz.reshape(1, HID)
    br_r = br.reshape(1, HID)
    bh_r = bh.reshape(1, HID)
    ba_r = ba.reshape(1, HID)
    ctxT = ctx.reshape(1, HID)
    WfT = Wf.reshape(1, HID)
    bf_r = bf.reshape(1, 1)

    deg0, deg1 = _spmm_call(ones32, srcr, dstr, ewr, zeros32)
    dis, xs = _prep(deg0, deg1, xflat)

    ax0, ax1 = _spmm_call(xs, srcr, dstr, ewr, zeros32)

    h, hs = _t0(ax0, ax1, dis, Wzx, bz_r, Whx, bh_r)
    hidden = [h]
    for t in range(1, T):
        p0, p1 = _spmm_call(hs, srcr, dstr, ewr, zeros32)
        z, rh, rhs = _make_gates(t)(p0, p1, h, ax0, ax1, dis,
                                    Wzx, Wzh, bz_r, Wrx, Wrh, br_r)
        q0, q1 = _spmm_call(rhs, srcr, dstr, ewr, zeros32)
        h, hs = _make_update(t)(q0, q1, rh, z, h, ax0, ax1, dis,
                                Whx, Whh, bh_r)
        hidden.append(h)

    return _attn(hidden, Wa, ba_r, ctxT, WfT, bf_r)
